# baseline (speedup 1.0000x reference)
import functools

import jax
import jax.numpy as jnp
from jax import lax
from jax.experimental import pallas as pl
from jax.experimental.pallas import tpu as pltpu
from jax.experimental.pallas import tpu_sc as plsc

NUM_EMB = 100000
SEM_IDS_DIM = 4
EMB_DIM = 64
N_SEM = 3
MAX_TAG = 1000
N_TAG = SEM_IDS_DIM - N_SEM
SEM_OFF = NUM_EMB * N_SEM
TOTAL_EMB = SEM_OFF + MAX_TAG * N_TAG + 1
PAD_IDX = TOTAL_EMB - 1
B, L = 4096, 200
LF = 4

NC = 2
NS = 16
NW = NC * NS
LANES = 16

RPC = 2
CHUNK = RPC * L
NB = 2
CUNROLL = 8

FUT_N = B * LF
ROWS_PER_W = B // NW
SEQ_CHUNKS = ROWS_PER_W // RPC
FUT_PER_W = FUT_N // NW


def _body(sem_seq, tok_seq, sem_fut, tok_fut, table, out_seq, out_fut,
          sem_v, tok_v, idx_v, rows_bf, rows_f32,
          fut_sem, fut_tok, fut_idx, fut_bf,
          id_sems, g_sems, w_sems):
    wid = lax.axis_index("s") * NC + lax.axis_index("c")
    row0w = wid * ROWS_PER_W
    tok_base0 = row0w * L
    fut_base = wid * FUT_PER_W
    ev2 = 2 * lax.iota(jnp.int32, 16)

    def idx_math(s, t):
        sem_c = jnp.minimum(jnp.maximum(s, 0), NUM_EMB - 1)
        tag_c = jnp.minimum(jnp.maximum(s, 0), MAX_TAG - 1)
        idx_sem = t * NUM_EMB + sem_c
        tag_layer = t - N_SEM
        idx_tag = jnp.where(
            tag_layer < N_TAG, SEM_OFF + tag_layer * MAX_TAG + tag_c, PAD_IDX
        )
        return jnp.where(t < N_SEM, idx_sem, idx_tag)

    def fire_ids(base, b):
        pltpu.async_copy(sem_seq.at[pl.ds(base, CHUNK)], sem_v.at[b],
                         id_sems.at[b])
        pltpu.async_copy(tok_seq.at[pl.ds(base, CHUNK)], tok_v.at[b],
                         id_sems.at[b])

    def wait_ids(b):
        pltpu.make_async_copy(sem_seq.at[pl.ds(0, CHUNK)], sem_v.at[b],
                              id_sems.at[b]).wait()
        pltpu.make_async_copy(tok_seq.at[pl.ds(0, CHUNK)], tok_v.at[b],
                              id_sems.at[b]).wait()

    def compute(b):
        for j in range(CHUNK // LANES):
            s = sem_v[b, pl.ds(j * LANES, LANES)]
            t = tok_v[b, pl.ds(j * LANES, LANES)]
            idx_v[b, pl.ds(j * LANES, LANES)] = idx_math(s, t)

    def fire_gathers(b):
        for j in range(3):
            pltpu.async_copy(table.at[idx_v.at[b, pl.ds(j * 128, 128)]],
                             rows_bf.at[b, pl.ds(j * 128, 128)],
                             g_sems.at[b])
        pltpu.async_copy(table.at[idx_v.at[b, pl.ds(384, 16)]],
                         rows_bf.at[b, pl.ds(384, 16)],
                         g_sems.at[b])

    def wait_gathers(b):
        for j in range(3):
            pltpu.make_async_copy(table.at[idx_v.at[b, pl.ds(j * 128, 128)]],
                                  rows_bf.at[b, pl.ds(j * 128, 128)],
                                  g_sems.at[b]).wait()
        pltpu.make_async_copy(table.at[idx_v.at[b, pl.ds(384, 16)]],
                              rows_bf.at[b, pl.ds(384, 16)],
                              g_sems.at[b]).wait()

    def convert(b):
        for r in range(RPC):
            def conv_iter(to, carry, r=r):
                for u in range(CUNROLL):
                    t2 = to * CUNROLL + u
                    rr = r * L + t2
                    t2v = jnp.broadcast_to(t2, (16,)).astype(jnp.int32)
                    for half in (0, 1):
                        src = rows_bf[b, rr, pl.ds(half * 32, 32)]
                        w = plsc.bitcast(src, jnp.int32)
                        lo = plsc.bitcast(w << 16, jnp.float32)
                        hi = plsc.bitcast(w & jnp.int32(-65536), jnp.float32)
                        pos = half * 32 + ev2
                        plsc.store_scatter(rows_f32.at[b, r], [t2v, pos], lo)
                        plsc.store_scatter(rows_f32.at[b, r], [t2v, pos + 1],
                                           hi)
                return carry

            lax.fori_loop(0, L // CUNROLL, conv_iter, 0)

    def fire_write(row0, b):
        pltpu.async_copy(rows_f32.at[b], out_seq.at[pl.ds(row0, RPC)],
                         w_sems.at[b])

    def wait_write(b):
        pltpu.make_async_copy(rows_f32.at[b], out_seq.at[pl.ds(0, RPC)],
                              w_sems.at[b]).wait()

    def fut_fire_ids():
        pltpu.async_copy(sem_fut.at[pl.ds(fut_base, FUT_PER_W)], fut_sem,
                         id_sems.at[0])
        pltpu.async_copy(tok_fut.at[pl.ds(fut_base, FUT_PER_W)], fut_tok,
                         id_sems.at[0])

    def fut_wait_ids():
        pltpu.make_async_copy(sem_fut.at[pl.ds(0, FUT_PER_W)], fut_sem,
                              id_sems.at[0]).wait()
        pltpu.make_async_copy(tok_fut.at[pl.ds(0, FUT_PER_W)], fut_tok,
                              id_sems.at[0]).wait()

    def fut_compute():
        for j in range(FUT_PER_W // LANES):
            s = fut_sem[pl.ds(j * LANES, LANES)]
            t = fut_tok[pl.ds(j * LANES, LANES)]
            fut_idx[j // 8, pl.ds((j % 8) * LANES, LANES)] = idx_math(s, t)

    def fut_fire_gathers():
        for j in range(4):
            pltpu.async_copy(table.at[fut_idx.at[j]],
                             fut_bf.at[pl.ds(j * 128, 128)], g_sems.at[0])

    def fut_wait_gathers():
        for j in range(4):
            pltpu.make_async_copy(table.at[fut_idx.at[j]],
                                  fut_bf.at[pl.ds(j * 128, 128)],
                                  g_sems.at[0]).wait()

    sbase = lambda i: tok_base0 + i * CHUNK
    srow = lambda i: row0w + i * RPC

    fire_ids(sbase(0), 0)
    fire_ids(sbase(1), 1)

    wait_ids(0)
    compute(0)
    fire_ids(sbase(2), 0)
    fire_gathers(0)

    wait_ids(1)
    compute(1)
    fire_ids(sbase(3), 1)
    fire_gathers(1)
    wait_gathers(0)
    convert(0)
    fire_write(srow(0), 0)

    def loop_body(g, carry):
        for b in (0, 1):
            i = 2 * g + b
            nb = 1 - b
            wait_ids(b)
            compute(b)
            fire_ids(sbase(i + 2), b)
            wait_write(b)
            fire_gathers(b)
            wait_gathers(nb)
            convert(nb)
            fire_write(srow(i - 1), nb)
        return carry

    lax.fori_loop(1, SEQ_CHUNKS // 2 - 1, loop_body, 0)

    wait_ids(0)
    compute(0)
    fut_fire_ids()
    wait_write(0)
    fire_gathers(0)
    wait_gathers(1)
    convert(1)
    fire_write(srow(SEQ_CHUNKS - 3), 1)

    wait_ids(1)
    compute(1)
    wait_write(1)
    fire_gathers(1)
    wait_gathers(0)
    convert(0)
    fire_write(srow(SEQ_CHUNKS - 2), 0)

    fut_wait_ids()
    fut_compute()
    fut_fire_gathers()
    wait_gathers(1)
    convert(1)
    fire_write(srow(SEQ_CHUNKS - 1), 1)

    fut_wait_gathers()
    wait_write(0)
    pltpu.async_copy(fut_bf, out_fut.at[pl.ds(fut_base, FUT_PER_W)],
                     w_sems.at[0])
    wait_write(1)
    pltpu.make_async_copy(fut_bf, out_fut.at[pl.ds(0, FUT_PER_W)],
                          w_sems.at[0]).wait()


@jax.jit
def _emb_lookup(sem_seq, tok_seq, sem_fut, tok_fut, table):
    mesh = plsc.VectorSubcoreMesh(core_axis_name="c", subcore_axis_name="s")
    f = pl.kernel(
        _body,
        out_type=(
            jax.ShapeDtypeStruct((B, L, EMB_DIM), jnp.float32),
            jax.ShapeDtypeStruct((FUT_N, EMB_DIM), jnp.bfloat16),
        ),
        mesh=mesh,
        scratch_types=[
            pltpu.VMEM((NB, CHUNK), jnp.int32),
            pltpu.VMEM((NB, CHUNK), jnp.int32),
            pltpu.VMEM((NB, 512), jnp.int32),
            pltpu.VMEM((NB, CHUNK, EMB_DIM), jnp.bfloat16),
            pltpu.VMEM((NB, RPC, L, EMB_DIM), jnp.float32),
            pltpu.VMEM((FUT_PER_W,), jnp.int32),
            pltpu.VMEM((FUT_PER_W,), jnp.int32),
            pltpu.VMEM((4, 128), jnp.int32),
            pltpu.VMEM((FUT_PER_W, EMB_DIM), jnp.bfloat16),
            pltpu.SemaphoreType.DMA((NB,)),
            pltpu.SemaphoreType.DMA((NB,)),
            pltpu.SemaphoreType.DMA((NB,)),
        ],
        compiler_params=pltpu.CompilerParams(
            use_tc_tiling_on_sc=False, needs_layout_passes=False
        ),
    )
    return f(sem_seq, tok_seq, sem_fut, tok_fut, table)


def kernel(sem_ids, token_type_ids, sem_ids_fut, token_type_ids_fut, emb_table):
    out_seq, out_fut = _emb_lookup(
        sem_ids.reshape(-1),
        token_type_ids.reshape(-1),
        sem_ids_fut.reshape(-1),
        token_type_ids_fut.reshape(-1),
        emb_table.astype(jnp.bfloat16),
    )
    return (
        out_seq,
        out_fut.astype(jnp.float32).reshape(B, LF, EMB_DIM),
    )
